# baseline (speedup 1.0000x reference)
import functools

import jax
import jax.numpy as jnp
from jax import lax
from jax.experimental import pallas as pl
from jax.experimental.pallas import tpu as pltpu
from jax.experimental.pallas import tpu_sc as plsc

N_NODES = 10000
N_PAD = 10240
N_EDGES = 320000
D_FEAT = 128
D_AUG = 144

N_CORES = 2
N_SUB = 16
CHUNK = 40

_EDGES_PER_CORE = N_EDGES // N_CORES
_EDGES_PER_TILE = _EDGES_PER_CORE // N_SUB
_CHUNKS_PER_TILE = _EDGES_PER_TILE // CHUNK
_ROWS_PER_TILE = N_PAD // N_SUB


def _make_agg(d):
    mesh = plsc.VectorSubcoreMesh(core_axis_name="c", subcore_axis_name="s")

    @functools.partial(
        pl.kernel,
        out_type=jax.ShapeDtypeStruct((N_CORES, N_PAD, d), jnp.float32),
        mesh=mesh,
        compiler_params=pltpu.CompilerParams(use_tc_tiling_on_sc=False),
        scratch_types=[
            pltpu.VMEM((_CHUNKS_PER_TILE, CHUNK), jnp.int32),
            [pltpu.VMEM((CHUNK,), jnp.int32) for _ in range(4)],
            [pltpu.VMEM((CHUNK, d), jnp.float32) for _ in range(4)],
            pltpu.VMEM_SHARED((N_PAD, d), jnp.float32),
            [pltpu.SemaphoreType.DMA for _ in range(4)],
            [pltpu.SemaphoreType.DMA for _ in range(4)],
            [pltpu.SemaphoreType.DMA for _ in range(4)],
        ],
    )
    def agg(table_hbm, src_hbm, dst_hbm, out_hbm, idx_s, idx_d,
            bufs, acc, gsems, ssems, isems):
        c = lax.axis_index("c")
        s = lax.axis_index("s")

        row_base = c * (_EDGES_PER_CORE // CHUNK) + s * _CHUNKS_PER_TILE
        pltpu.async_copy(src_hbm.at[pl.ds(row_base, _CHUNKS_PER_TILE)], idx_s, gsems[0])

        def ifetch(i, q):
            pltpu.async_copy(dst_hbm.at[row_base + i], idx_d[q], isems[q])

        def iwait(q):
            pltpu.make_async_copy(dst_hbm.at[row_base], idx_d[q], isems[q]).wait()

        def zrow(r, _):
            def zlane(j, _):
                bufs[0][r, pl.ds(j * 16, 16)] = jnp.zeros((16,), jnp.float32)
                return 0

            return lax.fori_loop(0, d // 16, zlane, 0)

        lax.fori_loop(0, CHUNK, zrow, 0)
        for k in range(_ROWS_PER_TILE // CHUNK):
            pltpu.async_copy(bufs[0], acc.at[pl.ds(s * _ROWS_PER_TILE + k * CHUNK, CHUNK)], ssems[0])
        for k in range(_ROWS_PER_TILE // CHUNK):
            pltpu.make_async_copy(bufs[0], acc.at[pl.ds(0, CHUNK)], ssems[0]).wait()
        pltpu.make_async_copy(src_hbm.at[pl.ds(row_base, _CHUNKS_PER_TILE)], idx_s, gsems[0]).wait()
        plsc.subcore_barrier()

        def gather(i, p):
            pltpu.async_copy(table_hbm.at[idx_s.at[i]], bufs[p], gsems[p])

        def gwait(p):
            pltpu.make_async_copy(table_hbm.at[idx_s.at[0]], bufs[p], gsems[p]).wait()

        def scat(i, p):
            pltpu.async_copy(bufs[p], acc.at[idx_d[p]], ssems[p], add=True)

        def swait(p):
            pltpu.make_async_copy(bufs[p], acc.at[idx_d[0]], ssems[p]).wait()

        for p in range(3):
            ifetch(p, p)
            gather(p, p)

        def chunk(i, _):
            def step(p):
                pn = (p + 3) % 4
                gwait(p)
                iwait(p)
                scat(i, p)

                @pl.when(i >= 1)
                def _():
                    swait(pn)

                @pl.when(i + 3 < _CHUNKS_PER_TILE)
                def _():
                    ifetch(i + 3, pn)
                    gather(i + 3, pn)

            for p in range(4):

                @pl.when(i % 4 == p)
                def _(p=p):
                    step(p)

            return 0

        lax.fori_loop(0, _CHUNKS_PER_TILE, chunk, 0)
        swait((_CHUNKS_PER_TILE - 1) % 4)
        plsc.subcore_barrier()

        pltpu.sync_copy(
            acc.at[pl.ds(s * _ROWS_PER_TILE, _ROWS_PER_TILE)],
            out_hbm.at[c, pl.ds(s * _ROWS_PER_TILE, _ROWS_PER_TILE)],
        )

    return agg


_agg_aug = _make_agg(D_AUG)
_agg_feat = _make_agg(D_FEAT)


def _tc1_body(pa_ref, pb_ref, x_ref, wl_ref, wr_ref, b_ref, h_ref, cnt_ref):
    agg = pa_ref[:, :D_FEAT] + pb_ref[:, :D_FEAT]
    cnt = jnp.maximum(pa_ref[:, D_FEAT:D_FEAT + 1] + pb_ref[:, D_FEAT:D_FEAT + 1], 1.0)
    mean = agg / cnt
    h = (
        jnp.dot(mean, wl_ref[:, :], precision=lax.Precision.HIGHEST)
        + jnp.dot(x_ref[:, :], wr_ref[:, :], precision=lax.Precision.HIGHEST)
        + b_ref[:][None, :]
    )
    h_ref[:, :] = jnp.maximum(h, 0.0)
    cnt_ref[:, :] = cnt


def _tc2_body(pa_ref, pb_ref, h_ref, cnt_ref, wl_ref, wr_ref, b_ref, out_ref):
    mean = (pa_ref[:, :] + pb_ref[:, :]) / cnt_ref[:, :]
    out_ref[:, :] = (
        jnp.dot(mean, wl_ref[:, :], precision=lax.Precision.HIGHEST)
        + jnp.dot(h_ref[:, :], wr_ref[:, :], precision=lax.Precision.HIGHEST)
        + b_ref[:][None, :]
    )


_ROW_BLK = 2000

_tc1 = pl.pallas_call(
    _tc1_body,
    grid=(N_NODES // _ROW_BLK,),
    in_specs=[
        pl.BlockSpec((_ROW_BLK, D_AUG), lambda i: (i, 0)),
        pl.BlockSpec((_ROW_BLK, D_AUG), lambda i: (i, 0)),
        pl.BlockSpec((_ROW_BLK, D_FEAT), lambda i: (i, 0)),
        pl.BlockSpec((D_FEAT, D_FEAT), lambda i: (0, 0)),
        pl.BlockSpec((D_FEAT, D_FEAT), lambda i: (0, 0)),
        pl.BlockSpec((D_FEAT,), lambda i: (0,)),
    ],
    out_specs=[
        pl.BlockSpec((_ROW_BLK, D_FEAT), lambda i: (i, 0)),
        pl.BlockSpec((_ROW_BLK, 1), lambda i: (i, 0)),
    ],
    out_shape=[
        jax.ShapeDtypeStruct((N_NODES, D_FEAT), jnp.float32),
        jax.ShapeDtypeStruct((N_NODES, 1), jnp.float32),
    ],
)

_tc2 = pl.pallas_call(
    _tc2_body,
    grid=(N_NODES // _ROW_BLK,),
    in_specs=[
        pl.BlockSpec((_ROW_BLK, D_FEAT), lambda i: (i, 0)),
        pl.BlockSpec((_ROW_BLK, D_FEAT), lambda i: (i, 0)),
        pl.BlockSpec((_ROW_BLK, D_FEAT), lambda i: (i, 0)),
        pl.BlockSpec((_ROW_BLK, 1), lambda i: (i, 0)),
        pl.BlockSpec((D_FEAT, D_FEAT), lambda i: (0, 0)),
        pl.BlockSpec((D_FEAT, D_FEAT), lambda i: (0, 0)),
        pl.BlockSpec((D_FEAT,), lambda i: (0,)),
    ],
    out_specs=pl.BlockSpec((_ROW_BLK, D_FEAT), lambda i: (i, 0)),
    out_shape=jax.ShapeDtypeStruct((N_NODES, D_FEAT), jnp.float32),
)


@jax.jit
def kernel(x, edge_index, W1_l, W1_r, b1, W2_l, W2_r, b2):
    src = edge_index[0].astype(jnp.int32).reshape(N_EDGES // CHUNK, CHUNK)
    dst = edge_index[1].astype(jnp.int32).reshape(N_EDGES // CHUNK, CHUNK)

    x_aug = jnp.concatenate(
        [x, jnp.ones((N_NODES, 1), jnp.float32), jnp.zeros((N_NODES, D_AUG - D_FEAT - 1), jnp.float32)],
        axis=1,
    )

    p1 = _agg_aug(x_aug, src, dst)
    h, cnt = _tc1(p1[0, :N_NODES], p1[1, :N_NODES], x, W1_l, W1_r, b1)

    p2 = _agg_feat(h, src, dst)
    out = _tc2(p2[0, :N_NODES], p2[1, :N_NODES], h, cnt, W2_l, W2_r, b2)
    return out


# speedup vs baseline: 1.0633x; 1.0633x over previous
import functools

import jax
import jax.numpy as jnp
from jax import lax
from jax.experimental import pallas as pl
from jax.experimental.pallas import tpu as pltpu
from jax.experimental.pallas import tpu_sc as plsc

N_NODES = 10000
N_PAD = 10112
N_EDGES = 320000
D_FEAT = 128
D_AUG = 144

N_CORES = 2
N_SUB = 16
CHUNK = 40

_EDGES_PER_CORE = N_EDGES // N_CORES
_EDGES_PER_TILE = _EDGES_PER_CORE // N_SUB
_CHUNKS_PER_TILE = _EDGES_PER_TILE // CHUNK
_ROWS_PER_TILE = N_PAD // N_SUB
_ZFULL = _ROWS_PER_TILE // CHUNK
_ZREM = _ROWS_PER_TILE - _ZFULL * CHUNK


def _make_agg(d):
    mesh = plsc.VectorSubcoreMesh(core_axis_name="c", subcore_axis_name="s")

    @functools.partial(
        pl.kernel,
        out_type=jax.ShapeDtypeStruct((N_CORES, N_PAD, d), jnp.float32),
        mesh=mesh,
        compiler_params=pltpu.CompilerParams(use_tc_tiling_on_sc=False),
        scratch_types=[
            pltpu.VMEM((_CHUNKS_PER_TILE, CHUNK), jnp.int32),
            [pltpu.VMEM((CHUNK,), jnp.int32) for _ in range(5)],
            [pltpu.VMEM((CHUNK, d), jnp.float32) for _ in range(5)],
            pltpu.VMEM_SHARED((N_PAD, d), jnp.float32),
            [pltpu.SemaphoreType.DMA for _ in range(5)],
            [pltpu.SemaphoreType.DMA for _ in range(5)],
            [pltpu.SemaphoreType.DMA for _ in range(5)],
        ],
    )
    def agg(table_hbm, src_hbm, dst_hbm, out_hbm, idx_s, idx_d,
            bufs, acc, gsems, ssems, isems):
        c = lax.axis_index("c")
        s = lax.axis_index("s")

        row_base = c * (_EDGES_PER_CORE // CHUNK) + s * _CHUNKS_PER_TILE
        pltpu.async_copy(src_hbm.at[pl.ds(row_base, _CHUNKS_PER_TILE)], idx_s, gsems[0])

        def ifetch(i, q):
            pltpu.async_copy(dst_hbm.at[row_base + i], idx_d[q], isems[q])

        def iwait(q):
            pltpu.make_async_copy(dst_hbm.at[row_base], idx_d[q], isems[q]).wait()

        def zrow(r, _):
            def zlane(j, _):
                bufs[0][r, pl.ds(j * 16, 16)] = jnp.zeros((16,), jnp.float32)
                return 0

            return lax.fori_loop(0, d // 16, zlane, 0)

        lax.fori_loop(0, CHUNK, zrow, 0)
        for k in range(_ZFULL):
            pltpu.async_copy(bufs[0], acc.at[pl.ds(s * _ROWS_PER_TILE + k * CHUNK, CHUNK)], ssems[0])
        if _ZREM:
            pltpu.async_copy(
                bufs[0].at[pl.ds(0, _ZREM)],
                acc.at[pl.ds(s * _ROWS_PER_TILE + _ZFULL * CHUNK, _ZREM)],
                ssems[1],
            )
        for k in range(_ZFULL):
            pltpu.make_async_copy(bufs[0], acc.at[pl.ds(0, CHUNK)], ssems[0]).wait()
        if _ZREM:
            pltpu.make_async_copy(bufs[0].at[pl.ds(0, _ZREM)], acc.at[pl.ds(0, _ZREM)], ssems[1]).wait()
        pltpu.make_async_copy(src_hbm.at[pl.ds(row_base, _CHUNKS_PER_TILE)], idx_s, gsems[0]).wait()
        plsc.subcore_barrier()

        def gather(i, p):
            pltpu.async_copy(table_hbm.at[idx_s.at[i]], bufs[p], gsems[p])

        def gwait(p):
            pltpu.make_async_copy(table_hbm.at[idx_s.at[0]], bufs[p], gsems[p]).wait()

        def scat(i, p):
            pltpu.async_copy(bufs[p], acc.at[idx_d[p]], ssems[p], add=True)

        def swait(p):
            pltpu.make_async_copy(bufs[p], acc.at[idx_d[0]], ssems[p]).wait()

        for p in range(4):
            ifetch(p, p)
            gather(p, p)

        def chunk(i, _):
            def step(p):
                pn = (p + 4) % 5
                gwait(p)
                iwait(p)
                scat(i, p)

                @pl.when(i >= 1)
                def _():
                    swait(pn)

                @pl.when(i + 4 < _CHUNKS_PER_TILE)
                def _():
                    ifetch(i + 4, pn)
                    gather(i + 4, pn)

            for p in range(5):

                @pl.when(i % 5 == p)
                def _(p=p):
                    step(p)

            return 0

        lax.fori_loop(0, _CHUNKS_PER_TILE, chunk, 0)
        swait((_CHUNKS_PER_TILE - 1) % 5)
        plsc.subcore_barrier()

        pltpu.sync_copy(
            acc.at[pl.ds(s * _ROWS_PER_TILE, _ROWS_PER_TILE)],
            out_hbm.at[c, pl.ds(s * _ROWS_PER_TILE, _ROWS_PER_TILE)],
        )

    return agg


_agg_aug = _make_agg(D_AUG)
_agg_feat = _make_agg(D_FEAT)


def _tc1_body(pa_ref, pb_ref, x_ref, wl_ref, wr_ref, b_ref, h_ref, cnt_ref):
    agg = pa_ref[:, :D_FEAT] + pb_ref[:, :D_FEAT]
    cnt = jnp.maximum(pa_ref[:, D_FEAT:D_FEAT + 1] + pb_ref[:, D_FEAT:D_FEAT + 1], 1.0)
    mean = agg / cnt
    h = (
        jnp.dot(mean, wl_ref[:, :], precision=lax.Precision.HIGHEST)
        + jnp.dot(x_ref[:, :], wr_ref[:, :], precision=lax.Precision.HIGHEST)
        + b_ref[:][None, :]
    )
    h_ref[:, :] = jnp.maximum(h, 0.0)
    cnt_ref[:, :] = cnt


def _tc2_body(pa_ref, pb_ref, h_ref, cnt_ref, wl_ref, wr_ref, b_ref, out_ref):
    mean = (pa_ref[:, :] + pb_ref[:, :]) / cnt_ref[:, :]
    out_ref[:, :] = (
        jnp.dot(mean, wl_ref[:, :], precision=lax.Precision.HIGHEST)
        + jnp.dot(h_ref[:, :], wr_ref[:, :], precision=lax.Precision.HIGHEST)
        + b_ref[:][None, :]
    )


_ROW_BLK = 2000

_tc1 = pl.pallas_call(
    _tc1_body,
    grid=(N_NODES // _ROW_BLK,),
    in_specs=[
        pl.BlockSpec((_ROW_BLK, D_AUG), lambda i: (i, 0)),
        pl.BlockSpec((_ROW_BLK, D_AUG), lambda i: (i, 0)),
        pl.BlockSpec((_ROW_BLK, D_FEAT), lambda i: (i, 0)),
        pl.BlockSpec((D_FEAT, D_FEAT), lambda i: (0, 0)),
        pl.BlockSpec((D_FEAT, D_FEAT), lambda i: (0, 0)),
        pl.BlockSpec((D_FEAT,), lambda i: (0,)),
    ],
    out_specs=[
        pl.BlockSpec((_ROW_BLK, D_FEAT), lambda i: (i, 0)),
        pl.BlockSpec((_ROW_BLK, 1), lambda i: (i, 0)),
    ],
    out_shape=[
        jax.ShapeDtypeStruct((N_NODES, D_FEAT), jnp.float32),
        jax.ShapeDtypeStruct((N_NODES, 1), jnp.float32),
    ],
)

_tc2 = pl.pallas_call(
    _tc2_body,
    grid=(N_NODES // _ROW_BLK,),
    in_specs=[
        pl.BlockSpec((_ROW_BLK, D_FEAT), lambda i: (i, 0)),
        pl.BlockSpec((_ROW_BLK, D_FEAT), lambda i: (i, 0)),
        pl.BlockSpec((_ROW_BLK, D_FEAT), lambda i: (i, 0)),
        pl.BlockSpec((_ROW_BLK, 1), lambda i: (i, 0)),
        pl.BlockSpec((D_FEAT, D_FEAT), lambda i: (0, 0)),
        pl.BlockSpec((D_FEAT, D_FEAT), lambda i: (0, 0)),
        pl.BlockSpec((D_FEAT,), lambda i: (0,)),
    ],
    out_specs=pl.BlockSpec((_ROW_BLK, D_FEAT), lambda i: (i, 0)),
    out_shape=jax.ShapeDtypeStruct((N_NODES, D_FEAT), jnp.float32),
)


@jax.jit
def kernel(x, edge_index, W1_l, W1_r, b1, W2_l, W2_r, b2):
    src = edge_index[0].astype(jnp.int32).reshape(N_EDGES // CHUNK, CHUNK)
    dst = edge_index[1].astype(jnp.int32).reshape(N_EDGES // CHUNK, CHUNK)

    x_aug = jnp.concatenate(
        [x, jnp.ones((N_NODES, 1), jnp.float32), jnp.zeros((N_NODES, D_AUG - D_FEAT - 1), jnp.float32)],
        axis=1,
    )

    p1 = _agg_aug(x_aug, src, dst)
    h, cnt = _tc1(p1[0, :N_NODES], p1[1, :N_NODES], x, W1_l, W1_r, b1)

    p2 = _agg_feat(h, src, dst)
    out = _tc2(p2[0, :N_NODES], p2[1, :N_NODES], h, cnt, W2_l, W2_r, b2)
    return out


# speedup vs baseline: 1.1382x; 1.0705x over previous
import functools

import jax
import jax.numpy as jnp
from jax import lax
from jax.experimental import pallas as pl
from jax.experimental.pallas import tpu as pltpu
from jax.experimental.pallas import tpu_sc as plsc

N_NODES = 10000
N_PAD = 10112
N_EDGES = 320000
D_FEAT = 128
D_AUG = 144

N_CORES = 2
N_SUB = 16
CHUNK = 40

_EDGES_PER_CORE = N_EDGES // N_CORES
_EDGES_PER_TILE = _EDGES_PER_CORE // N_SUB
_CHUNKS_PER_TILE = _EDGES_PER_TILE // CHUNK
_ROWS_PER_TILE = N_PAD // N_SUB
_ZFULL = _ROWS_PER_TILE // CHUNK
_ZREM = _ROWS_PER_TILE - _ZFULL * CHUNK


def _make_agg(d):
    mesh = plsc.VectorSubcoreMesh(core_axis_name="c", subcore_axis_name="s")

    @functools.partial(
        pl.kernel,
        out_type=jax.ShapeDtypeStruct((N_CORES, N_PAD, d), jnp.float32),
        mesh=mesh,
        compiler_params=pltpu.CompilerParams(use_tc_tiling_on_sc=False),
        scratch_types=[
            pltpu.VMEM((_CHUNKS_PER_TILE, CHUNK), jnp.int32),
            [pltpu.VMEM((CHUNK,), jnp.int32) for _ in range(5)],
            [pltpu.VMEM((CHUNK, d), jnp.float32) for _ in range(5)],
            pltpu.VMEM_SHARED((N_PAD, d), jnp.float32),
            [pltpu.SemaphoreType.DMA for _ in range(5)],
            [pltpu.SemaphoreType.DMA for _ in range(5)],
            [pltpu.SemaphoreType.DMA for _ in range(5)],
        ],
    )
    def agg(table_hbm, src_hbm, dst_hbm, out_hbm, idx_s, idx_d,
            bufs, acc, gsems, ssems, isems):
        c = lax.axis_index("c")
        s = lax.axis_index("s")

        row_base = c * (_EDGES_PER_CORE // CHUNK) + s * _CHUNKS_PER_TILE
        pltpu.async_copy(src_hbm.at[pl.ds(row_base, _CHUNKS_PER_TILE)], idx_s, gsems[0])

        def ifetch(i, q):
            pltpu.async_copy(dst_hbm.at[row_base + i], idx_d[q], isems[q])

        def iwait(q):
            pltpu.make_async_copy(dst_hbm.at[row_base], idx_d[q], isems[q]).wait()

        def zrow(r, _):
            def zlane(j, _):
                bufs[0][r, pl.ds(j * 16, 16)] = jnp.zeros((16,), jnp.float32)
                return 0

            return lax.fori_loop(0, d // 16, zlane, 0)

        lax.fori_loop(0, CHUNK, zrow, 0)
        for k in range(_ZFULL):
            pltpu.async_copy(bufs[0], acc.at[pl.ds(s * _ROWS_PER_TILE + k * CHUNK, CHUNK)], ssems[0])
        if _ZREM:
            pltpu.async_copy(
                bufs[0].at[pl.ds(0, _ZREM)],
                acc.at[pl.ds(s * _ROWS_PER_TILE + _ZFULL * CHUNK, _ZREM)],
                ssems[1],
            )
        for k in range(_ZFULL):
            pltpu.make_async_copy(bufs[0], acc.at[pl.ds(0, CHUNK)], ssems[0]).wait()
        if _ZREM:
            pltpu.make_async_copy(bufs[0].at[pl.ds(0, _ZREM)], acc.at[pl.ds(0, _ZREM)], ssems[1]).wait()
        pltpu.make_async_copy(src_hbm.at[pl.ds(row_base, _CHUNKS_PER_TILE)], idx_s, gsems[0]).wait()
        plsc.subcore_barrier()

        def gather(i, p):
            pltpu.async_copy(table_hbm.at[idx_s.at[i]], bufs[p], gsems[p])

        def gwait(p):
            pltpu.make_async_copy(table_hbm.at[idx_s.at[0]], bufs[p], gsems[p]).wait()

        def scat(i, p):
            pltpu.async_copy(bufs[p], acc.at[idx_d[p]], ssems[p], add=True)

        def swait(p):
            pltpu.make_async_copy(bufs[p], acc.at[idx_d[0]], ssems[p]).wait()

        for p in range(4):
            ifetch(p, p)
            gather(p, p)

        def chunk(i, _):
            def step(p):
                pn = (p + 4) % 5
                gwait(p)
                iwait(p)
                scat(i, p)

                @pl.when(i >= 1)
                def _():
                    swait(pn)

                @pl.when(i + 4 < _CHUNKS_PER_TILE)
                def _():
                    ifetch(i + 4, pn)
                    gather(i + 4, pn)

            for p in range(5):

                @pl.when(i % 5 == p)
                def _(p=p):
                    step(p)

            return 0

        lax.fori_loop(0, _CHUNKS_PER_TILE, chunk, 0)
        swait((_CHUNKS_PER_TILE - 1) % 5)
        plsc.subcore_barrier()

        pltpu.sync_copy(
            acc.at[pl.ds(s * _ROWS_PER_TILE, _ROWS_PER_TILE)],
            out_hbm.at[c, pl.ds(s * _ROWS_PER_TILE, _ROWS_PER_TILE)],
        )

    return agg


_agg_aug = _make_agg(D_AUG)
_agg_feat = _make_agg(D_FEAT)


def _tc1_body(p_ref, x_ref, wl_ref, wr_ref, b_ref, h_ref, cnt_ref):
    pa = p_ref[0]
    pb = p_ref[1]
    agg = pa[:, :D_FEAT] + pb[:, :D_FEAT]
    cnt = jnp.maximum(pa[:, D_FEAT:D_FEAT + 1] + pb[:, D_FEAT:D_FEAT + 1], 1.0)
    mean = agg / cnt
    h = (
        jnp.dot(mean, wl_ref[:, :], precision=lax.Precision.HIGHEST)
        + jnp.dot(x_ref[:, :], wr_ref[:, :], precision=lax.Precision.HIGHEST)
        + b_ref[:][None, :]
    )
    h_ref[:, :] = jnp.maximum(h, 0.0)
    cnt_ref[:, :] = cnt


def _tc2_body(p_ref, h_ref, cnt_ref, wl_ref, wr_ref, b_ref, out_ref):
    mean = (p_ref[0] + p_ref[1]) / cnt_ref[:, :]
    out_ref[:, :] = (
        jnp.dot(mean, wl_ref[:, :], precision=lax.Precision.HIGHEST)
        + jnp.dot(h_ref[:, :], wr_ref[:, :], precision=lax.Precision.HIGHEST)
        + b_ref[:][None, :]
    )


_ROW_BLK = 2000

_tc1 = pl.pallas_call(
    _tc1_body,
    grid=(N_NODES // _ROW_BLK,),
    in_specs=[
        pl.BlockSpec((N_CORES, _ROW_BLK, D_AUG), lambda i: (0, i, 0)),
        pl.BlockSpec((_ROW_BLK, D_FEAT), lambda i: (i, 0)),
        pl.BlockSpec((D_FEAT, D_FEAT), lambda i: (0, 0)),
        pl.BlockSpec((D_FEAT, D_FEAT), lambda i: (0, 0)),
        pl.BlockSpec((D_FEAT,), lambda i: (0,)),
    ],
    out_specs=[
        pl.BlockSpec((_ROW_BLK, D_FEAT), lambda i: (i, 0)),
        pl.BlockSpec((_ROW_BLK, 1), lambda i: (i, 0)),
    ],
    out_shape=[
        jax.ShapeDtypeStruct((N_NODES, D_FEAT), jnp.float32),
        jax.ShapeDtypeStruct((N_NODES, 1), jnp.float32),
    ],
)

_tc2 = pl.pallas_call(
    _tc2_body,
    grid=(N_NODES // _ROW_BLK,),
    in_specs=[
        pl.BlockSpec((N_CORES, _ROW_BLK, D_FEAT), lambda i: (0, i, 0)),
        pl.BlockSpec((_ROW_BLK, D_FEAT), lambda i: (i, 0)),
        pl.BlockSpec((_ROW_BLK, 1), lambda i: (i, 0)),
        pl.BlockSpec((D_FEAT, D_FEAT), lambda i: (0, 0)),
        pl.BlockSpec((D_FEAT, D_FEAT), lambda i: (0, 0)),
        pl.BlockSpec((D_FEAT,), lambda i: (0,)),
    ],
    out_specs=pl.BlockSpec((_ROW_BLK, D_FEAT), lambda i: (i, 0)),
    out_shape=jax.ShapeDtypeStruct((N_NODES, D_FEAT), jnp.float32),
)


@jax.jit
def kernel(x, edge_index, W1_l, W1_r, b1, W2_l, W2_r, b2):
    src = edge_index[0].astype(jnp.int32).reshape(N_EDGES // CHUNK, CHUNK)
    dst = edge_index[1].astype(jnp.int32).reshape(N_EDGES // CHUNK, CHUNK)

    x_aug = jnp.concatenate(
        [x, jnp.ones((N_NODES, 1), jnp.float32), jnp.zeros((N_NODES, D_AUG - D_FEAT - 1), jnp.float32)],
        axis=1,
    )

    p1 = _agg_aug(x_aug, src, dst)
    h, cnt = _tc1(p1, x, W1_l, W1_r, b1)

    p2 = _agg_feat(h, src, dst)
    out = _tc2(p2, h, cnt, W2_l, W2_r, b2)
    return out


# speedup vs baseline: 1.1529x; 1.0129x over previous
import functools

import jax
import jax.numpy as jnp
from jax import lax
from jax.experimental import pallas as pl
from jax.experimental.pallas import tpu as pltpu
from jax.experimental.pallas import tpu_sc as plsc

N_NODES = 10000
N_PAD = 10112
N_EDGES = 320000
D_FEAT = 128
D_AUG = 144

N_CORES = 2
N_SUB = 16
CHUNK = 40

_EDGES_PER_CORE = N_EDGES // N_CORES
_EDGES_PER_TILE = _EDGES_PER_CORE // N_SUB
_CHUNKS_PER_TILE = _EDGES_PER_TILE // CHUNK
_ROWS_PER_TILE = N_PAD // N_SUB
_ZFULL = _ROWS_PER_TILE // CHUNK
_ZREM = _ROWS_PER_TILE - _ZFULL * CHUNK


def _make_agg(d):
    mesh = plsc.VectorSubcoreMesh(core_axis_name="c", subcore_axis_name="s")

    @functools.partial(
        pl.kernel,
        out_type=jax.ShapeDtypeStruct((N_CORES, N_PAD, d), jnp.float32),
        mesh=mesh,
        compiler_params=pltpu.CompilerParams(use_tc_tiling_on_sc=False),
        scratch_types=[
            pltpu.VMEM((_CHUNKS_PER_TILE, CHUNK), jnp.int32),
            [pltpu.VMEM((CHUNK,), jnp.int32) for _ in range(5)],
            [pltpu.VMEM((CHUNK, d), jnp.float32) for _ in range(5)],
            pltpu.VMEM_SHARED((N_PAD, d), jnp.float32),
            [pltpu.SemaphoreType.DMA for _ in range(5)],
            [pltpu.SemaphoreType.DMA for _ in range(5)],
            [pltpu.SemaphoreType.DMA for _ in range(5)],
        ],
    )
    def agg(table_hbm, src_hbm, dst_hbm, out_hbm, idx_s, idx_d,
            bufs, acc, gsems, ssems, isems):
        c = lax.axis_index("c")
        s = lax.axis_index("s")

        row_base = c * (_EDGES_PER_CORE // CHUNK) + s * _CHUNKS_PER_TILE
        pltpu.async_copy(src_hbm.at[pl.ds(row_base, _CHUNKS_PER_TILE)], idx_s, gsems[0])

        def ifetch(i, q):
            pltpu.async_copy(dst_hbm.at[row_base + i], idx_d[q], isems[q])

        def iwait(q):
            pltpu.make_async_copy(dst_hbm.at[row_base], idx_d[q], isems[q]).wait()

        def zrow(r, _):
            def zlane(j, _):
                bufs[0][r, pl.ds(j * 16, 16)] = jnp.zeros((16,), jnp.float32)
                return 0

            return lax.fori_loop(0, d // 16, zlane, 0)

        lax.fori_loop(0, CHUNK, zrow, 0)
        for k in range(_ZFULL):
            pltpu.async_copy(bufs[0], acc.at[pl.ds(s * _ROWS_PER_TILE + k * CHUNK, CHUNK)], ssems[0])
        if _ZREM:
            pltpu.async_copy(
                bufs[0].at[pl.ds(0, _ZREM)],
                acc.at[pl.ds(s * _ROWS_PER_TILE + _ZFULL * CHUNK, _ZREM)],
                ssems[1],
            )
        for k in range(_ZFULL):
            pltpu.make_async_copy(bufs[0], acc.at[pl.ds(0, CHUNK)], ssems[0]).wait()
        if _ZREM:
            pltpu.make_async_copy(bufs[0].at[pl.ds(0, _ZREM)], acc.at[pl.ds(0, _ZREM)], ssems[1]).wait()
        pltpu.make_async_copy(src_hbm.at[pl.ds(row_base, _CHUNKS_PER_TILE)], idx_s, gsems[0]).wait()
        plsc.subcore_barrier()

        def gather(i, p):
            pltpu.async_copy(table_hbm.at[idx_s.at[i]], bufs[p], gsems[p])

        def gwait(p):
            pltpu.make_async_copy(table_hbm.at[idx_s.at[0]], bufs[p], gsems[p]).wait()

        def scat(i, p):
            pltpu.async_copy(bufs[p], acc.at[idx_d[p]], ssems[p], add=True)

        def swait(p):
            pltpu.make_async_copy(bufs[p], acc.at[idx_d[0]], ssems[p]).wait()

        for p in range(4):
            ifetch(p, p)
            gather(p, p)

        def chunk(i, _):
            def step(p):
                pn = (p + 4) % 5
                gwait(p)
                iwait(p)
                scat(i, p)

                @pl.when(i >= 1)
                def _():
                    swait(pn)

                @pl.when(i + 4 < _CHUNKS_PER_TILE)
                def _():
                    ifetch(i + 4, pn)
                    gather(i + 4, pn)

            for p in range(5):

                @pl.when(i % 5 == p)
                def _(p=p):
                    step(p)

            return 0

        lax.fori_loop(0, _CHUNKS_PER_TILE, chunk, 0)
        swait((_CHUNKS_PER_TILE - 1) % 5)
        plsc.subcore_barrier()

        pltpu.sync_copy(
            acc.at[pl.ds(s * _ROWS_PER_TILE, _ROWS_PER_TILE)],
            out_hbm.at[c, pl.ds(s * _ROWS_PER_TILE, _ROWS_PER_TILE)],
        )

    return agg


_agg_aug = _make_agg(D_AUG)
_agg_feat = _make_agg(D_FEAT)


def _tcr_body(x_ref, wr_ref, b_ref, o_ref):
    o_ref[:, :] = (
        jnp.dot(x_ref[:, :], wr_ref[:, :], precision=lax.Precision.HIGHEST)
        + b_ref[:][None, :]
    )


def _tc1_body(p_ref, xr_ref, wl_ref, h_ref, cnt_ref):
    pa = p_ref[0]
    pb = p_ref[1]
    agg = pa[:, :D_FEAT] + pb[:, :D_FEAT]
    cnt = jnp.maximum(pa[:, D_FEAT:D_FEAT + 1] + pb[:, D_FEAT:D_FEAT + 1], 1.0)
    mean = agg / cnt
    h = jnp.dot(mean, wl_ref[:, :], precision=lax.Precision.HIGHEST) + xr_ref[:, :]
    h_ref[:, :] = jnp.maximum(h, 0.0)
    cnt_ref[:, :] = cnt


def _tc2_body(p_ref, xr_ref, cnt_ref, wl_ref, out_ref):
    mean = (p_ref[0] + p_ref[1]) / cnt_ref[:, :]
    out_ref[:, :] = (
        jnp.dot(mean, wl_ref[:, :], precision=lax.Precision.HIGHEST) + xr_ref[:, :]
    )


_ROW_BLK = 2000

_tcr = pl.pallas_call(
    _tcr_body,
    grid=(N_NODES // _ROW_BLK,),
    in_specs=[
        pl.BlockSpec((_ROW_BLK, D_FEAT), lambda i: (i, 0)),
        pl.BlockSpec((D_FEAT, D_FEAT), lambda i: (0, 0)),
        pl.BlockSpec((D_FEAT,), lambda i: (0,)),
    ],
    out_specs=pl.BlockSpec((_ROW_BLK, D_FEAT), lambda i: (i, 0)),
    out_shape=jax.ShapeDtypeStruct((N_NODES, D_FEAT), jnp.float32),
)

_tc1 = pl.pallas_call(
    _tc1_body,
    grid=(N_NODES // _ROW_BLK,),
    in_specs=[
        pl.BlockSpec((N_CORES, _ROW_BLK, D_AUG), lambda i: (0, i, 0)),
        pl.BlockSpec((_ROW_BLK, D_FEAT), lambda i: (i, 0)),
        pl.BlockSpec((D_FEAT, D_FEAT), lambda i: (0, 0)),
    ],
    out_specs=[
        pl.BlockSpec((_ROW_BLK, D_FEAT), lambda i: (i, 0)),
        pl.BlockSpec((_ROW_BLK, 1), lambda i: (i, 0)),
    ],
    out_shape=[
        jax.ShapeDtypeStruct((N_NODES, D_FEAT), jnp.float32),
        jax.ShapeDtypeStruct((N_NODES, 1), jnp.float32),
    ],
)

_tc2 = pl.pallas_call(
    _tc2_body,
    grid=(N_NODES // _ROW_BLK,),
    in_specs=[
        pl.BlockSpec((N_CORES, _ROW_BLK, D_FEAT), lambda i: (0, i, 0)),
        pl.BlockSpec((_ROW_BLK, D_FEAT), lambda i: (i, 0)),
        pl.BlockSpec((_ROW_BLK, 1), lambda i: (i, 0)),
        pl.BlockSpec((D_FEAT, D_FEAT), lambda i: (0, 0)),
    ],
    out_specs=pl.BlockSpec((_ROW_BLK, D_FEAT), lambda i: (i, 0)),
    out_shape=jax.ShapeDtypeStruct((N_NODES, D_FEAT), jnp.float32),
)


@jax.jit
def kernel(x, edge_index, W1_l, W1_r, b1, W2_l, W2_r, b2):
    src = edge_index[0].astype(jnp.int32).reshape(N_EDGES // CHUNK, CHUNK)
    dst = edge_index[1].astype(jnp.int32).reshape(N_EDGES // CHUNK, CHUNK)

    x_aug = jnp.concatenate(
        [x, jnp.ones((N_NODES, 1), jnp.float32), jnp.zeros((N_NODES, D_AUG - D_FEAT - 1), jnp.float32)],
        axis=1,
    )

    p1 = _agg_aug(x_aug, src, dst)
    xr1 = _tcr(x, W1_r, b1)
    h, cnt = _tc1(p1, xr1, W1_l)

    p2 = _agg_feat(h, src, dst)
    xr2 = _tcr(h, W2_r, b2)
    out = _tc2(p2, xr2, cnt, W2_l)
    return out
